# baseline (speedup 1.0000x reference)
import functools

import jax
import jax.numpy as jnp
from jax import lax
from jax.experimental import pallas as pl
from jax.experimental.pallas import tpu as pltpu
from jax.experimental.pallas import tpu_sc as plsc

E = 16
C = 256
K = 512
S = 512
B = 8
T = 262144
VOCAB = 257
NTOK = B * T
EP = E // 2
NWIN = B * (T // S)
KE = K * E

NC = 2
NS = 16
NW = NC * NS
PER_W = NTOK // NW
CHUNK = 1024
NCHUNK = PER_W // CHUNK

BM = 512
GRID = NWIN // BM


def _sc_gather_body(x_hbm, table_hbm, out_hbm, xbuf, tbl, rows0, rows1,
                    isem, tsem, osem0, osem1):
    wid = lax.axis_index("s") * NC + lax.axis_index("c")
    base0 = wid * PER_W
    rows = (rows0, rows1)
    osem = (osem0, osem1)

    cp_t = pltpu.async_copy(table_hbm, tbl, tsem)
    cp_x = pltpu.async_copy(x_hbm.at[pl.ds(base0, PER_W)], xbuf, isem)
    cp_t.wait()
    cp_x.wait()

    def fill(c, b):
        rbuf = rows[b]

        def grp(g, carry):
            for gg in range(2):
                g2 = g * 2 + gg
                idx16 = xbuf[pl.ds(c * CHUNK + g2 * 16, 16)]
                base = g2 * (16 * EP)
                for e2 in range(EP):
                    vals = plsc.load_gather(tbl, [idx16 + e2 * VOCAB])
                    rbuf[pl.ds(base + e2 * 16, 16)] = vals
            return carry

        lax.fori_loop(0, CHUNK // 32, grp, 0)

    def out_start(c, b):
        return pltpu.async_copy(
            rows[b], out_hbm.at[pl.ds((base0 + c * CHUNK) * EP, CHUNK * EP)],
            osem[b])

    def out_wait(c, b):
        pltpu.make_async_copy(
            rows[b], out_hbm.at[pl.ds((base0 + c * CHUNK) * EP, CHUNK * EP)],
            osem[b]).wait()

    fill(0, 0)
    out_start(0, 0)
    fill(1, 1)
    out_start(1, 1)

    def lap(g, carry):
        for b in range(2):
            c = 2 * g + b
            out_wait(c - 2, b)
            fill(c, b)
            out_start(c, b)
        return carry

    lax.fori_loop(1, NCHUNK // 2, lap, 0)
    for b in range(2):
        out_wait(NCHUNK - 2 + b, b)


@functools.cache
def _sc_gather():
    return pl.kernel(
        _sc_gather_body,
        out_type=jax.ShapeDtypeStruct((NTOK * EP,), jnp.int32),
        mesh=plsc.VectorSubcoreMesh(core_axis_name="c", subcore_axis_name="s"),
        scratch_types=[
            pltpu.VMEM((PER_W,), jnp.int32),
            pltpu.VMEM((VOCAB * EP,), jnp.int32),
            pltpu.VMEM((CHUNK * EP,), jnp.int32),
            pltpu.VMEM((CHUNK * EP,), jnp.int32),
            pltpu.SemaphoreType.DMA,
            pltpu.SemaphoreType.DMA,
            pltpu.SemaphoreType.DMA,
            pltpu.SemaphoreType.DMA,
        ],
        compiler_params=pltpu.CompilerParams(use_tc_tiling_on_sc=False,
                                             needs_layout_passes=False),
    )


def _tc_body(z_ref, w_ref, b_ref, ws_ref, bs_ref, gw_ref, gb_ref,
             f1w_ref, f1b_ref, f2w_ref, f2b_ref, out_ref, m1_ref, m2_ref):
    i = pl.program_id(0)

    @pl.when(i == 0)
    def _init():
        m1_ref[...] = jnp.full((B, C), -jnp.inf, jnp.float32)
        m2_ref[...] = jnp.full((B, C), -jnp.inf, jnp.float32)

    a = z_ref[...]
    c = jnp.dot(a, w_ref[...], preferred_element_type=jnp.float32)
    c = c + b_ref[...]
    u = c[:, :C] * jax.nn.sigmoid(c[:, C:2 * C])
    s = jnp.dot(u, ws_ref[...], preferred_element_type=jnp.float32)
    s = s + bs_ref[...]
    s = jnp.where(s >= 0.0, s, 0.01 * s)
    v = c[:, 2 * C:3 * C] * jax.nn.sigmoid(c[:, 3 * C:])

    m1_blk = jnp.max(s, axis=0, keepdims=True)
    m2_blk = jnp.max(v, axis=0, keepdims=True)
    row = lax.broadcasted_iota(jnp.int32, (B, 1), 0)
    sel = row == i
    m1_ref[...] = jnp.where(sel, jnp.maximum(m1_ref[...], m1_blk), m1_ref[...])
    m2_ref[...] = jnp.where(sel, jnp.maximum(m2_ref[...], m2_blk), m2_ref[...])

    @pl.when(i == pl.num_programs(0) - 1)
    def _head():
        gates = jax.nn.sigmoid(
            jnp.dot(m1_ref[...], gw_ref[...],
                    preferred_element_type=jnp.float32) + gb_ref[...])
        pooled = m2_ref[...] * gates
        f = jnp.dot(pooled, f1w_ref[...], preferred_element_type=jnp.float32)
        f = jnp.maximum(f + f1b_ref[...], 0.0)
        o = jnp.dot(f, f2w_ref[...], preferred_element_type=jnp.float32)
        out_ref[...] = o + f2b_ref[...]


def _full(shape):
    return pl.BlockSpec(shape, lambda i: (0, 0))


_tc_call = pl.pallas_call(
    _tc_body,
    grid=(GRID,),
    in_specs=[
        pl.BlockSpec((BM, KE), lambda i: (i, 0)),
        _full((KE, 4 * C)),
        _full((1, 4 * C)),
        _full((C, C)),
        _full((1, C)),
        _full((C, C)),
        _full((1, C)),
        _full((C, C)),
        _full((1, C)),
        _full((C, 128)),
        _full((1, 128)),
    ],
    out_specs=pl.BlockSpec((B, 128), lambda i: (0, 0)),
    out_shape=jax.ShapeDtypeStruct((B, 128), jnp.float32),
    scratch_shapes=[
        pltpu.VMEM((B, C), jnp.float32),
        pltpu.VMEM((B, C), jnp.float32),
    ],
)


def kernel(x, embed, ctx_conv_w, ctx_conv_b, ctx_share_w, ctx_share_b,
           gcg_conv_w, gcg_conv_b, gate_w, gate_b,
           fc1_w, fc1_b, fc2_w, fc2_b):
    tblp = lax.bitcast_convert_type(
        embed.astype(jnp.bfloat16).reshape(VOCAB, EP, 2).transpose(1, 0, 2),
        jnp.int32).reshape(EP * VOCAB)

    zf = _sc_gather()(x.reshape(NTOK), tblp)
    z = lax.bitcast_convert_type(zf, jnp.bfloat16).reshape(NWIN, KE)

    def _wprep(w):
        return w.astype(jnp.bfloat16).reshape(
            2 * C, EP, 2, K // 16, 16).transpose(3, 1, 4, 2, 0).reshape(
                KE, 2 * C)

    w_all = jnp.concatenate([_wprep(ctx_conv_w), _wprep(gcg_conv_w)], axis=1)
    b_all = jnp.concatenate([ctx_conv_b, gcg_conv_b])[None, :]
    ws = ctx_share_w[:, :, 0].T
    bs = ctx_share_b[None, :]
    gw = gate_w.T
    gb = gate_b[None, :]
    f1w = fc1_w.T
    f1b = fc1_b[None, :]
    f2w = jnp.pad(fc2_w.T, ((0, 0), (0, 128 - fc2_w.shape[0])))
    f2b = jnp.pad(fc2_b, (0, 128 - fc2_b.shape[0]))[None, :]

    out = _tc_call(z, w_all, b_all, ws, bs, gw, gb, f1w, f1b, f2w, f2b)
    return out[:, :fc2_w.shape[0]]


# speedup vs baseline: 2.1371x; 2.1371x over previous
import functools

import jax
import jax.numpy as jnp
from jax import lax
from jax.experimental import pallas as pl
from jax.experimental.pallas import tpu as pltpu
from jax.experimental.pallas import tpu_sc as plsc

E = 16
C = 256
K = 512
S = 512
B = 8
T = 262144
VOCAB = 257
NTOK = B * T
EP = E // 2
NWIN = B * (T // S)
KE = K * E

NC = 2
NS = 16
NW = NC * NS
PER_W = NTOK // NW
CHUNK = 1024
NCHUNK = PER_W // CHUNK

BM = 512
GRID = NWIN // BM


def _sc_gather_body(x_hbm, table_hbm, out_hbm, xbuf, tbl, rows0, rows1,
                    isem, tsem, osem0, osem1):
    wid = lax.axis_index("s") * NC + lax.axis_index("c")
    base0 = wid * PER_W
    rows = (rows0, rows1)
    osem = (osem0, osem1)

    cp_t = pltpu.async_copy(table_hbm, tbl, tsem)
    cp_x = pltpu.async_copy(x_hbm.at[pl.ds(base0, PER_W)], xbuf, isem)
    cp_t.wait()
    cp_x.wait()

    def fill(c, b):
        rbuf = rows[b]

        def grp(g, carry):
            for gg in range(2):
                g2 = g * 2 + gg
                idx16 = xbuf[pl.ds(c * CHUNK + g2 * 16, 16)]
                base = g2 * (16 * EP)
                for e2 in range(EP):
                    vals = plsc.load_gather(tbl, [idx16 + e2 * VOCAB])
                    rbuf[pl.ds(base + e2 * 16, 16)] = vals
            return carry

        lax.fori_loop(0, CHUNK // 32, grp, 0)

    def out_start(c, b):
        return pltpu.async_copy(
            rows[b], out_hbm.at[pl.ds((base0 + c * CHUNK) * EP, CHUNK * EP)],
            osem[b])

    def out_wait(c, b):
        pltpu.make_async_copy(
            rows[b], out_hbm.at[pl.ds((base0 + c * CHUNK) * EP, CHUNK * EP)],
            osem[b]).wait()

    fill(0, 0)
    out_start(0, 0)
    fill(1, 1)
    out_start(1, 1)

    def lap(g, carry):
        for b in range(2):
            c = 2 * g + b
            out_wait(c - 2, b)
            fill(c, b)
            out_start(c, b)
        return carry

    lax.fori_loop(1, NCHUNK // 2, lap, 0)
    for b in range(2):
        out_wait(NCHUNK - 2 + b, b)


@functools.cache
def _sc_gather():
    return pl.kernel(
        _sc_gather_body,
        out_type=jax.ShapeDtypeStruct((NTOK * EP,), jnp.int32),
        mesh=plsc.VectorSubcoreMesh(core_axis_name="c", subcore_axis_name="s"),
        scratch_types=[
            pltpu.VMEM((PER_W,), jnp.int32),
            pltpu.VMEM((VOCAB * EP,), jnp.int32),
            pltpu.VMEM((CHUNK * EP,), jnp.int32),
            pltpu.VMEM((CHUNK * EP,), jnp.int32),
            pltpu.SemaphoreType.DMA,
            pltpu.SemaphoreType.DMA,
            pltpu.SemaphoreType.DMA,
            pltpu.SemaphoreType.DMA,
        ],
        compiler_params=pltpu.CompilerParams(use_tc_tiling_on_sc=False,
                                             needs_layout_passes=False),
    )


def _tc_body(z_ref, w_ref, b_ref, ws_ref, bs_ref, gw_ref, gb_ref,
             f1w_ref, f1b_ref, f2w_ref, f2b_ref, out_ref, m1_ref, m2_ref):
    i = pl.program_id(0)

    @pl.when(i == 0)
    def _init():
        m1_ref[...] = jnp.full((B, C), -jnp.inf, jnp.float32)
        m2_ref[...] = jnp.full((B, C), -jnp.inf, jnp.float32)

    ai = z_ref[...]
    alo = lax.bitcast_convert_type(ai << 16, jnp.float32).astype(jnp.bfloat16)
    ahi = lax.bitcast_convert_type(
        ai & jnp.int32(-65536), jnp.float32).astype(jnp.bfloat16)
    c = (jnp.dot(alo, w_ref[0], preferred_element_type=jnp.float32) +
         jnp.dot(ahi, w_ref[1], preferred_element_type=jnp.float32))
    c = c + b_ref[...]
    u = c[:, :C] * jax.nn.sigmoid(c[:, C:2 * C])
    s = jnp.dot(u, ws_ref[...], preferred_element_type=jnp.float32)
    s = s + bs_ref[...]
    s = jnp.where(s >= 0.0, s, 0.01 * s)
    v = c[:, 2 * C:3 * C] * jax.nn.sigmoid(c[:, 3 * C:])

    m1_blk = jnp.max(s, axis=0, keepdims=True)
    m2_blk = jnp.max(v, axis=0, keepdims=True)
    row = lax.broadcasted_iota(jnp.int32, (B, 1), 0)
    sel = row == i
    m1_ref[...] = jnp.where(sel, jnp.maximum(m1_ref[...], m1_blk), m1_ref[...])
    m2_ref[...] = jnp.where(sel, jnp.maximum(m2_ref[...], m2_blk), m2_ref[...])

    @pl.when(i == pl.num_programs(0) - 1)
    def _head():
        gates = jax.nn.sigmoid(
            jnp.dot(m1_ref[...], gw_ref[...],
                    preferred_element_type=jnp.float32) + gb_ref[...])
        pooled = m2_ref[...] * gates
        f = jnp.dot(pooled, f1w_ref[...], preferred_element_type=jnp.float32)
        f = jnp.maximum(f + f1b_ref[...], 0.0)
        o = jnp.dot(f, f2w_ref[...], preferred_element_type=jnp.float32)
        out_ref[...] = o + f2b_ref[...]


def _full(shape):
    return pl.BlockSpec(shape, lambda i: (0, 0))


_tc_call = pl.pallas_call(
    _tc_body,
    grid=(GRID,),
    in_specs=[
        pl.BlockSpec((BM, KE // 2), lambda i: (i, 0)),
        pl.BlockSpec((2, KE // 2, 4 * C), lambda i: (0, 0, 0)),
        _full((1, 4 * C)),
        _full((C, C)),
        _full((1, C)),
        _full((C, C)),
        _full((1, C)),
        _full((C, C)),
        _full((1, C)),
        _full((C, 128)),
        _full((1, 128)),
    ],
    out_specs=pl.BlockSpec((B, 128), lambda i: (0, 0)),
    out_shape=jax.ShapeDtypeStruct((B, 128), jnp.float32),
    scratch_shapes=[
        pltpu.VMEM((B, C), jnp.float32),
        pltpu.VMEM((B, C), jnp.float32),
    ],
)


def kernel(x, embed, ctx_conv_w, ctx_conv_b, ctx_share_w, ctx_share_b,
           gcg_conv_w, gcg_conv_b, gate_w, gate_b,
           fc1_w, fc1_b, fc2_w, fc2_b):
    tblp = lax.bitcast_convert_type(
        embed.astype(jnp.bfloat16).reshape(VOCAB, EP, 2).transpose(1, 0, 2),
        jnp.int32).reshape(EP * VOCAB)

    zf = _sc_gather()(x.reshape(NTOK), tblp)
    z = zf.reshape(NWIN, KE // 2)

    def _wprep(w):
        return w.astype(jnp.bfloat16).reshape(
            2 * C, EP, 2, K // 16, 16).transpose(2, 3, 1, 4, 0).reshape(
                2, KE // 2, 2 * C)

    w_all = jnp.concatenate([_wprep(ctx_conv_w), _wprep(gcg_conv_w)], axis=2)
    b_all = jnp.concatenate([ctx_conv_b, gcg_conv_b])[None, :]
    ws = ctx_share_w[:, :, 0].T
    bs = ctx_share_b[None, :]
    gw = gate_w.T
    gb = gate_b[None, :]
    f1w = fc1_w.T
    f1b = fc1_b[None, :]
    f2w = jnp.pad(fc2_w.T, ((0, 0), (0, 128 - fc2_w.shape[0])))
    f2b = jnp.pad(fc2_b, (0, 128 - fc2_b.shape[0]))[None, :]

    out = _tc_call(z, w_all, b_all, ws, bs, gw, gb, f1w, f1b, f2w, f2b)
    return out[:, :fc2_w.shape[0]]


# speedup vs baseline: 2.2537x; 1.0545x over previous
import functools

import jax
import jax.numpy as jnp
from jax import lax
from jax.experimental import pallas as pl
from jax.experimental.pallas import tpu as pltpu
from jax.experimental.pallas import tpu_sc as plsc

E = 16
C = 256
K = 512
S = 512
B = 8
T = 262144
VOCAB = 257
NTOK = B * T
EP = E // 2
NWIN = B * (T // S)
KE = K * E

BH = B // 2
NTOK_H = NTOK // 2
NWIN_H = NWIN // 2

NC = 2
NS = 16
NW = NC * NS
CHUNK = 1024

BM = 512
GRID = NWIN_H // BM


@functools.cache
def _sc_gather(ntok):
    per_w = ntok // NW
    nchunk = per_w // CHUNK

    def body(x_hbm, table_hbm, out_hbm, xbuf, tbl, rows0, rows1,
             isem, tsem, osem0, osem1):
        wid = lax.axis_index("s") * NC + lax.axis_index("c")
        base0 = wid * per_w
        rows = (rows0, rows1)
        osem = (osem0, osem1)

        cp_t = pltpu.async_copy(table_hbm, tbl, tsem)
        cp_x = pltpu.async_copy(x_hbm.at[pl.ds(base0, per_w)], xbuf, isem)
        cp_t.wait()
        cp_x.wait()

        def fill(c, b):
            rbuf = rows[b]

            def grp(g, carry):
                for gg in range(2):
                    g2 = g * 2 + gg
                    idx16 = xbuf[pl.ds(c * CHUNK + g2 * 16, 16)]
                    base = g2 * (16 * EP)
                    for e2 in range(EP):
                        vals = plsc.load_gather(tbl, [idx16 + e2 * VOCAB])
                        rbuf[pl.ds(base + e2 * 16, 16)] = vals
                return carry

            lax.fori_loop(0, CHUNK // 32, grp, 0)

        def out_start(c, b):
            return pltpu.async_copy(
                rows[b],
                out_hbm.at[pl.ds((base0 + c * CHUNK) * EP, CHUNK * EP)],
                osem[b])

        def out_wait(c, b):
            pltpu.make_async_copy(
                rows[b],
                out_hbm.at[pl.ds((base0 + c * CHUNK) * EP, CHUNK * EP)],
                osem[b]).wait()

        fill(0, 0)
        out_start(0, 0)
        fill(1, 1)
        out_start(1, 1)

        def lap(g, carry):
            for b in range(2):
                c = 2 * g + b
                out_wait(c - 2, b)
                fill(c, b)
                out_start(c, b)
            return carry

        lax.fori_loop(1, nchunk // 2, lap, 0)
        for b in range(2):
            out_wait(nchunk - 2 + b, b)

    return pl.kernel(
        body,
        out_type=jax.ShapeDtypeStruct((ntok * EP,), jnp.int32),
        mesh=plsc.VectorSubcoreMesh(core_axis_name="c", subcore_axis_name="s"),
        scratch_types=[
            pltpu.VMEM((per_w,), jnp.int32),
            pltpu.VMEM((VOCAB * EP,), jnp.int32),
            pltpu.VMEM((CHUNK * EP,), jnp.int32),
            pltpu.VMEM((CHUNK * EP,), jnp.int32),
            pltpu.SemaphoreType.DMA,
            pltpu.SemaphoreType.DMA,
            pltpu.SemaphoreType.DMA,
            pltpu.SemaphoreType.DMA,
        ],
        compiler_params=pltpu.CompilerParams(use_tc_tiling_on_sc=False,
                                             needs_layout_passes=False),
    )


def _tc_body(z_ref, w_ref, b_ref, ws_ref, bs_ref, m1_ref, m2_ref):
    i = pl.program_id(0)

    @pl.when(i == 0)
    def _init():
        m1_ref[...] = jnp.full((BH, C), -jnp.inf, jnp.float32)
        m2_ref[...] = jnp.full((BH, C), -jnp.inf, jnp.float32)

    ai = z_ref[...]
    alo = lax.bitcast_convert_type(ai << 16, jnp.float32).astype(jnp.bfloat16)
    ahi = lax.bitcast_convert_type(
        ai & jnp.int32(-65536), jnp.float32).astype(jnp.bfloat16)
    c = (jnp.dot(alo, w_ref[0], preferred_element_type=jnp.float32) +
         jnp.dot(ahi, w_ref[1], preferred_element_type=jnp.float32))
    c = c + b_ref[...]
    u = c[:, :C] * jax.nn.sigmoid(c[:, C:2 * C])
    s = jnp.dot(u, ws_ref[...], preferred_element_type=jnp.float32)
    s = s + bs_ref[...]
    s = jnp.where(s >= 0.0, s, 0.01 * s)
    v = c[:, 2 * C:3 * C] * jax.nn.sigmoid(c[:, 3 * C:])

    m1_blk = jnp.max(s, axis=0, keepdims=True)
    m2_blk = jnp.max(v, axis=0, keepdims=True)
    row = lax.broadcasted_iota(jnp.int32, (BH, 1), 0)
    sel = row == i
    m1_ref[...] = jnp.where(sel, jnp.maximum(m1_ref[...], m1_blk), m1_ref[...])
    m2_ref[...] = jnp.where(sel, jnp.maximum(m2_ref[...], m2_blk), m2_ref[...])


def _full(shape):
    return pl.BlockSpec(shape, lambda i: (0,) * len(shape))


_tc_call = pl.pallas_call(
    _tc_body,
    grid=(GRID,),
    in_specs=[
        pl.BlockSpec((BM, KE // 2), lambda i: (i, 0)),
        pl.BlockSpec((2, KE // 2, 4 * C), lambda i: (0, 0, 0)),
        _full((1, 4 * C)),
        _full((C, C)),
        _full((1, C)),
    ],
    out_specs=[_full((BH, C)), _full((BH, C))],
    out_shape=[jax.ShapeDtypeStruct((BH, C), jnp.float32),
               jax.ShapeDtypeStruct((BH, C), jnp.float32)],
)


def _head_body(m1a_ref, m2a_ref, m1b_ref, m2b_ref, gw_ref, gb_ref,
               f1w_ref, f1b_ref, f2w_ref, f2b_ref, out_ref):
    m1 = jnp.concatenate([m1a_ref[...], m1b_ref[...]], axis=0)
    m2 = jnp.concatenate([m2a_ref[...], m2b_ref[...]], axis=0)
    gates = jax.nn.sigmoid(
        jnp.dot(m1, gw_ref[...], preferred_element_type=jnp.float32)
        + gb_ref[...])
    pooled = m2 * gates
    f = jnp.dot(pooled, f1w_ref[...], preferred_element_type=jnp.float32)
    f = jnp.maximum(f + f1b_ref[...], 0.0)
    o = jnp.dot(f, f2w_ref[...], preferred_element_type=jnp.float32)
    out_ref[...] = o + f2b_ref[...]


_head_call = pl.pallas_call(
    _head_body,
    out_shape=jax.ShapeDtypeStruct((B, 128), jnp.float32),
)


def kernel(x, embed, ctx_conv_w, ctx_conv_b, ctx_share_w, ctx_share_b,
           gcg_conv_w, gcg_conv_b, gate_w, gate_b,
           fc1_w, fc1_b, fc2_w, fc2_b):
    tblp = lax.bitcast_convert_type(
        embed.astype(jnp.bfloat16).reshape(VOCAB, EP, 2).transpose(1, 0, 2),
        jnp.int32).reshape(EP * VOCAB)

    def _wprep(w):
        return w.astype(jnp.bfloat16).reshape(
            2 * C, EP, 2, K // 16, 16).transpose(2, 3, 1, 4, 0).reshape(
                2, KE // 2, 2 * C)

    w_all = jnp.concatenate([_wprep(ctx_conv_w), _wprep(gcg_conv_w)], axis=2)
    b_all = jnp.concatenate([ctx_conv_b, gcg_conv_b])[None, :]
    ws = ctx_share_w[:, :, 0].T
    bs = ctx_share_b[None, :]
    gw = gate_w.T
    gb = gate_b[None, :]
    f1w = fc1_w.T
    f1b = fc1_b[None, :]
    f2w = jnp.pad(fc2_w.T, ((0, 0), (0, 128 - fc2_w.shape[0])))
    f2b = jnp.pad(fc2_b, (0, 128 - fc2_b.shape[0]))[None, :]

    gather = _sc_gather(NTOK_H)
    xf = x.reshape(2, NTOK_H)
    z1 = gather(xf[0], tblp).reshape(NWIN_H, KE // 2)
    z2 = gather(xf[1], tblp).reshape(NWIN_H, KE // 2)
    m1a, m2a = _tc_call(z1, w_all, b_all, ws, bs)
    m1b, m2b = _tc_call(z2, w_all, b_all, ws, bs)

    out = _head_call(m1a, m2a, m1b, m2b, gw, gb, f1w, f1b, f2w, f2b)
    return out[:, :fc2_w.shape[0]]


# speedup vs baseline: 2.4797x; 1.1003x over previous
import functools

import jax
import jax.numpy as jnp
from jax import lax
from jax.experimental import pallas as pl
from jax.experimental.pallas import tpu as pltpu
from jax.experimental.pallas import tpu_sc as plsc

E = 16
C = 256
K = 512
S = 512
B = 8
T = 262144
VOCAB = 257
NTOK = B * T
EP = E // 2
NWIN = B * (T // S)
KE = K * E

NSPLIT = 4
BH = B // NSPLIT
NTOK_H = NTOK // NSPLIT
NWIN_H = NWIN // NSPLIT

NC = 2
NS = 16
NW = NC * NS
CHUNK = 1024

BM = 512
GRID = NWIN_H // BM


@functools.cache
def _sc_gather(ntok):
    per_w = ntok // NW
    nchunk = per_w // CHUNK

    def body(x_hbm, table_hbm, out_hbm, xbuf, tbl, rows0, rows1,
             isem, tsem, osem0, osem1):
        wid = lax.axis_index("s") * NC + lax.axis_index("c")
        base0 = wid * per_w
        rows = (rows0, rows1)
        osem = (osem0, osem1)

        cp_t = pltpu.async_copy(table_hbm, tbl, tsem)
        cp_x = pltpu.async_copy(x_hbm.at[pl.ds(base0, per_w)], xbuf, isem)
        cp_t.wait()
        cp_x.wait()

        def fill(c, b):
            rbuf = rows[b]

            def grp(g, carry):
                for gg in range(2):
                    g2 = g * 2 + gg
                    idx16 = xbuf[pl.ds(c * CHUNK + g2 * 16, 16)]
                    base = g2 * (16 * EP)
                    for e2 in range(EP):
                        vals = plsc.load_gather(tbl, [idx16 + e2 * VOCAB])
                        rbuf[pl.ds(base + e2 * 16, 16)] = vals
                return carry

            lax.fori_loop(0, CHUNK // 32, grp, 0)

        def out_start(c, b):
            return pltpu.async_copy(
                rows[b],
                out_hbm.at[pl.ds((base0 + c * CHUNK) * EP, CHUNK * EP)],
                osem[b])

        def out_wait(c, b):
            pltpu.make_async_copy(
                rows[b],
                out_hbm.at[pl.ds((base0 + c * CHUNK) * EP, CHUNK * EP)],
                osem[b]).wait()

        fill(0, 0)
        out_start(0, 0)
        fill(1, 1)
        out_start(1, 1)

        def lap(g, carry):
            for b in range(2):
                c = 2 * g + b
                out_wait(c - 2, b)
                fill(c, b)
                out_start(c, b)
            return carry

        lax.fori_loop(1, nchunk // 2, lap, 0)
        for b in range(2):
            out_wait(nchunk - 2 + b, b)

    return pl.kernel(
        body,
        out_type=jax.ShapeDtypeStruct((ntok * EP,), jnp.int32),
        mesh=plsc.VectorSubcoreMesh(core_axis_name="c", subcore_axis_name="s"),
        scratch_types=[
            pltpu.VMEM((per_w,), jnp.int32),
            pltpu.VMEM((VOCAB * EP,), jnp.int32),
            pltpu.VMEM((CHUNK * EP,), jnp.int32),
            pltpu.VMEM((CHUNK * EP,), jnp.int32),
            pltpu.SemaphoreType.DMA,
            pltpu.SemaphoreType.DMA,
            pltpu.SemaphoreType.DMA,
            pltpu.SemaphoreType.DMA,
        ],
        compiler_params=pltpu.CompilerParams(use_tc_tiling_on_sc=False,
                                             needs_layout_passes=False),
    )


def _tc_body(z_ref, w_ref, b_ref, ws_ref, bs_ref, m1_ref, m2_ref):
    i = pl.program_id(0)

    @pl.when(i == 0)
    def _init():
        m1_ref[...] = jnp.full((BH, C), -jnp.inf, jnp.float32)
        m2_ref[...] = jnp.full((BH, C), -jnp.inf, jnp.float32)

    ai = z_ref[...]
    alo = lax.bitcast_convert_type(ai << 16, jnp.float32).astype(jnp.bfloat16)
    ahi = lax.bitcast_convert_type(
        ai & jnp.int32(-65536), jnp.float32).astype(jnp.bfloat16)
    c = (jnp.dot(alo, w_ref[0], preferred_element_type=jnp.float32) +
         jnp.dot(ahi, w_ref[1], preferred_element_type=jnp.float32))
    c = c + b_ref[...]
    u = c[:, :C] * jax.nn.sigmoid(c[:, C:2 * C])
    s = jnp.dot(u, ws_ref[...], preferred_element_type=jnp.float32)
    s = s + bs_ref[...]
    s = jnp.where(s >= 0.0, s, 0.01 * s)
    v = c[:, 2 * C:3 * C] * jax.nn.sigmoid(c[:, 3 * C:])

    m1_blk = jnp.max(s, axis=0, keepdims=True)
    m2_blk = jnp.max(v, axis=0, keepdims=True)
    row = lax.broadcasted_iota(jnp.int32, (BH, 1), 0)
    sel = row == i
    m1_ref[...] = jnp.where(sel, jnp.maximum(m1_ref[...], m1_blk), m1_ref[...])
    m2_ref[...] = jnp.where(sel, jnp.maximum(m2_ref[...], m2_blk), m2_ref[...])


def _full(shape):
    return pl.BlockSpec(shape, lambda i: (0,) * len(shape))


_tc_call = pl.pallas_call(
    _tc_body,
    grid=(GRID,),
    in_specs=[
        pl.BlockSpec((BM, KE // 2), lambda i: (i, 0)),
        pl.BlockSpec((2, KE // 2, 4 * C), lambda i: (0, 0, 0)),
        _full((1, 4 * C)),
        _full((C, C)),
        _full((1, C)),
    ],
    out_specs=[_full((BH, C)), _full((BH, C))],
    out_shape=[jax.ShapeDtypeStruct((BH, C), jnp.float32),
               jax.ShapeDtypeStruct((BH, C), jnp.float32)],
)


def _head_body(*refs):
    m_refs, (gw_ref, gb_ref, f1w_ref, f1b_ref, f2w_ref, f2b_ref,
             out_ref) = refs[:2 * NSPLIT], refs[2 * NSPLIT:]
    m1 = jnp.concatenate([r[...] for r in m_refs[0::2]], axis=0)
    m2 = jnp.concatenate([r[...] for r in m_refs[1::2]], axis=0)
    gates = jax.nn.sigmoid(
        jnp.dot(m1, gw_ref[...], preferred_element_type=jnp.float32)
        + gb_ref[...])
    pooled = m2 * gates
    f = jnp.dot(pooled, f1w_ref[...], preferred_element_type=jnp.float32)
    f = jnp.maximum(f + f1b_ref[...], 0.0)
    o = jnp.dot(f, f2w_ref[...], preferred_element_type=jnp.float32)
    out_ref[...] = o + f2b_ref[...]


_head_call = pl.pallas_call(
    _head_body,
    out_shape=jax.ShapeDtypeStruct((B, 128), jnp.float32),
)


def kernel(x, embed, ctx_conv_w, ctx_conv_b, ctx_share_w, ctx_share_b,
           gcg_conv_w, gcg_conv_b, gate_w, gate_b,
           fc1_w, fc1_b, fc2_w, fc2_b):
    tblp = lax.bitcast_convert_type(
        embed.astype(jnp.bfloat16).reshape(VOCAB, EP, 2).transpose(1, 0, 2),
        jnp.int32).reshape(EP * VOCAB)

    def _wprep(w):
        return w.astype(jnp.bfloat16).reshape(
            2 * C, EP, 2, K // 16, 16).transpose(2, 3, 1, 4, 0).reshape(
                2, KE // 2, 2 * C)

    w_all = jnp.concatenate([_wprep(ctx_conv_w), _wprep(gcg_conv_w)], axis=2)
    b_all = jnp.concatenate([ctx_conv_b, gcg_conv_b])[None, :]
    ws = ctx_share_w[:, :, 0].T
    bs = ctx_share_b[None, :]
    gw = gate_w.T
    gb = gate_b[None, :]
    f1w = fc1_w.T
    f1b = fc1_b[None, :]
    f2w = jnp.pad(fc2_w.T, ((0, 0), (0, 128 - fc2_w.shape[0])))
    f2b = jnp.pad(fc2_b, (0, 128 - fc2_b.shape[0]))[None, :]

    gather = _sc_gather(NTOK_H)
    xf = x.reshape(NSPLIT, NTOK_H)
    zs = [gather(xf[i], tblp).reshape(NWIN_H, KE // 2)
          for i in range(NSPLIT)]
    ms = []
    for z in zs:
        ms.extend(_tc_call(z, w_all, b_all, ws, bs))

    out = _head_call(*ms, gw, gb, f1w, f1b, f2w, f2b)
    return out[:, :fc2_w.shape[0]]
